# baseline (speedup 1.0000x reference)
import functools

import jax
import jax.numpy as jnp
from jax import lax
from jax.experimental import pallas as pl
from jax.experimental.pallas import tpu as pltpu
from jax.experimental.pallas import tpu_sc as plsc

_N = 16384
_C = 1000
_WPAD = 1024
_NC, _NS, _L = 2, 16, 16
_NW = _NC * _NS
_PER_W = _N // _NW
_CHUNK = 128
_NCHUNK = _PER_W // _CHUNK

_mesh = plsc.VectorSubcoreMesh(core_axis_name="c", subcore_axis_name="s")


@functools.partial(
    pl.kernel,
    out_type=jax.ShapeDtypeStruct((_NW, _L), jnp.float32),
    mesh=_mesh,
    scratch_types=[
        pltpu.VMEM((_PER_W,), jnp.int32),
        pltpu.VMEM((_PER_W,), jnp.int32),
        pltpu.VMEM((_PER_W,), jnp.float32),
        pltpu.VMEM((_PER_W,), jnp.float32),
        pltpu.VMEM((_L,), jnp.float32),
        pltpu.SemaphoreType.DMA,
    ],
)
def _nll_partials(prob_hbm, tgt_hbm, w_hbm, out_hbm,
                  tgt_v, idx_v, gat_v, wgat_v, acc_v, sem):
    cid = lax.axis_index("c")
    sid = lax.axis_index("s")
    wid = sid * _NC + cid
    base = wid * _PER_W

    pltpu.sync_copy(tgt_hbm.at[pl.ds(base, _PER_W)], tgt_v)

    lane = lax.iota(jnp.int32, 16)

    def idx_body(j, _):
        t = tgt_v[pl.ds(j * _L, _L)]
        row = base + j * _L + lane
        idx_v[pl.ds(j * _L, _L)] = row * _C + t
        return 0

    lax.fori_loop(0, _PER_W // _L, idx_body, 0)

    copies = []
    for q in range(_NCHUNK):
        copies.append(pltpu.async_copy(
            prob_hbm.at[idx_v.at[pl.ds(q * _CHUNK, _CHUNK)]],
            gat_v.at[pl.ds(q * _CHUNK, _CHUNK)],
            sem,
        ))
        copies.append(pltpu.async_copy(
            w_hbm.at[tgt_v.at[pl.ds(q * _CHUNK, _CHUNK)]],
            wgat_v.at[pl.ds(q * _CHUNK, _CHUNK)],
            sem,
        ))
    for c in copies:
        c.wait()

    def sum_body(j, acc):
        p = gat_v[pl.ds(j * _L, _L)]
        w = wgat_v[pl.ds(j * _L, _L)]
        return acc + p * w

    acc = lax.fori_loop(0, _PER_W // _L, sum_body,
                        jnp.zeros((_L,), jnp.float32))
    acc_v[...] = acc
    pltpu.sync_copy(acc_v, out_hbm.at[wid])


def kernel(prob, target, weight):
    w_pad = jnp.zeros((_WPAD,), jnp.float32).at[:_C].set(weight)
    partials = _nll_partials(prob.reshape(-1), target, w_pad)
    return -jnp.sum(partials)


# speedup vs baseline: 1.3748x; 1.3748x over previous
import functools

import jax
import jax.numpy as jnp
from jax import lax
from jax.experimental import pallas as pl
from jax.experimental.pallas import tpu as pltpu
from jax.experimental.pallas import tpu_sc as plsc

_N = 16384
_C = 1000
_WPAD = 1024
_NC, _NS, _L = 2, 16, 16
_NW = _NC * _NS
_PER_W = _N // _NW
_RPC = 32
_NCHUNK = _PER_W // _RPC
_WCH = 128

_mesh = plsc.VectorSubcoreMesh(core_axis_name="c", subcore_axis_name="s")


@functools.partial(
    pl.kernel,
    out_type=jax.ShapeDtypeStruct((_NW, _L), jnp.float32),
    mesh=_mesh,
    compiler_params=pltpu.CompilerParams(use_tc_tiling_on_sc=True,
                                         needs_layout_passes=False),
    scratch_types=[
        pltpu.VMEM((_PER_W,), jnp.int32),
        pltpu.VMEM((_PER_W,), jnp.float32),
        pltpu.VMEM((_RPC, _C), jnp.float32),
        pltpu.VMEM((_RPC, _C), jnp.float32),
        pltpu.VMEM((_L,), jnp.float32),
        pltpu.SemaphoreType.DMA,
        pltpu.SemaphoreType.DMA,
    ],
)
def _nll_partials(prob_hbm, tgt_hbm, w_hbm, out_hbm,
                  tgt_v, wgat_v, buf0, buf1, acc_v, sem0, sem1):
    cid = lax.axis_index("c")
    sid = lax.axis_index("s")
    wid = sid * _NC + cid
    base = wid * _PER_W

    pltpu.sync_copy(tgt_hbm.at[pl.ds(base, _PER_W)], tgt_v)
    wcopies = []
    for q in range(_PER_W // _WCH):
        wcopies.append(pltpu.async_copy(
            w_hbm.at[tgt_v.at[pl.ds(q * _WCH, _WCH)]],
            wgat_v.at[pl.ds(q * _WCH, _WCH)],
            sem0,
        ))
    for c in wcopies:
        c.wait()

    bufs = (buf0, buf1)
    sems = (sem0, sem1)
    copies = [None, None]
    copies[0] = pltpu.async_copy(
        prob_hbm.at[pl.ds(base, _RPC), :], bufs[0], sems[0])

    lane = lax.iota(jnp.int32, 16)
    acc = jnp.zeros((_L,), jnp.float32)
    for k in range(_NCHUNK):
        cur = k % 2
        nxt = 1 - cur
        if k + 1 < _NCHUNK:
            copies[nxt] = pltpu.async_copy(
                prob_hbm.at[pl.ds(base + (k + 1) * _RPC, _RPC), :],
                bufs[nxt], sems[nxt])
        copies[cur].wait()
        for j in range(_RPC // _L):
            off = k * _RPC + j * _L
            t = tgt_v[pl.ds(off, _L)]
            w = wgat_v[pl.ds(off, _L)]
            rows = j * _L + lane
            g = plsc.load_gather(bufs[cur], [rows, t])
            acc = acc + g * w

    acc_v[...] = acc
    pltpu.sync_copy(acc_v, out_hbm.at[wid])


def kernel(prob, target, weight):
    w_pad = jnp.zeros((_WPAD,), jnp.float32).at[:_C].set(weight)
    partials = _nll_partials(prob, target, w_pad)
    return -jnp.sum(partials)


# speedup vs baseline: 1.4181x; 1.0315x over previous
import functools

import jax
import jax.numpy as jnp
from jax import lax
from jax.experimental import pallas as pl
from jax.experimental.pallas import tpu as pltpu
from jax.experimental.pallas import tpu_sc as plsc

_N = 16384
_C = 1000
_WPAD = 1024
_NC, _NS, _L = 2, 16, 16
_NW = _NC * _NS

_N_SC = 4096
_N_TC = _N - _N_SC
_PER_W = _N_SC // _NW
_RPC = 32
_NCHUNK = _PER_W // _RPC
_WCH = 128

_BR = 512
_NB_TC = _N_TC // _BR

_mesh = plsc.VectorSubcoreMesh(core_axis_name="c", subcore_axis_name="s")


@functools.partial(
    pl.kernel,
    out_type=jax.ShapeDtypeStruct((_NW, _L), jnp.float32),
    mesh=_mesh,
    compiler_params=pltpu.CompilerParams(use_tc_tiling_on_sc=True,
                                         needs_layout_passes=False),
    scratch_types=[
        pltpu.VMEM((_PER_W,), jnp.int32),
        pltpu.VMEM((_PER_W,), jnp.float32),
        pltpu.VMEM((_RPC, _C), jnp.float32),
        pltpu.VMEM((_RPC, _C), jnp.float32),
        pltpu.VMEM((_L,), jnp.float32),
        pltpu.SemaphoreType.DMA,
        pltpu.SemaphoreType.DMA,
    ],
)
def _nll_sc_partials(prob_hbm, tgt_hbm, w_hbm, out_hbm,
                     tgt_v, wgat_v, buf0, buf1, acc_v, sem0, sem1):
    cid = lax.axis_index("c")
    sid = lax.axis_index("s")
    wid = sid * _NC + cid
    base = wid * _PER_W

    pltpu.sync_copy(tgt_hbm.at[pl.ds(base, _PER_W)], tgt_v)
    wcopies = []
    for q in range(_PER_W // _WCH):
        wcopies.append(pltpu.async_copy(
            w_hbm.at[tgt_v.at[pl.ds(q * _WCH, _WCH)]],
            wgat_v.at[pl.ds(q * _WCH, _WCH)],
            sem0,
        ))
    for c in wcopies:
        c.wait()

    bufs = (buf0, buf1)
    sems = (sem0, sem1)
    copies = [None, None]
    copies[0] = pltpu.async_copy(
        prob_hbm.at[pl.ds(base, _RPC), :], bufs[0], sems[0])

    lane = lax.iota(jnp.int32, 16)
    acc = jnp.zeros((_L,), jnp.float32)
    for k in range(_NCHUNK):
        cur = k % 2
        nxt = 1 - cur
        if k + 1 < _NCHUNK:
            copies[nxt] = pltpu.async_copy(
                prob_hbm.at[pl.ds(base + (k + 1) * _RPC, _RPC), :],
                bufs[nxt], sems[nxt])
        copies[cur].wait()
        for j in range(_RPC // _L):
            off = k * _RPC + j * _L
            t = tgt_v[pl.ds(off, _L)]
            w = wgat_v[pl.ds(off, _L)]
            rows = j * _L + lane
            g = plsc.load_gather(bufs[cur], [rows, t])
            acc = acc + g * w

    acc_v[...] = acc
    pltpu.sync_copy(acc_v, out_hbm.at[wid])


def _nll_tc_block(prob_ref, tgt_ref, w_ref, out_ref):
    t = tgt_ref[0, 0, :]
    col = lax.broadcasted_iota(jnp.int32, (_BR, _C), 1)
    mask = col == t[:, None]
    pw = prob_ref[...] * w_ref[...]
    out_ref[...] = jnp.sum(jnp.where(mask, pw, 0.0)).reshape(1, 1, 1)


_B0 = _N_SC // _BR

_nll_tc_partials = pl.pallas_call(
    _nll_tc_block,
    grid=(_NB_TC,),
    in_specs=[
        pl.BlockSpec((_BR, _C), lambda i: (i + _B0, 0)),
        pl.BlockSpec((1, 1, _BR), lambda i: (i + _B0, 0, 0)),
        pl.BlockSpec((1, _C), lambda i: (0, 0)),
    ],
    out_specs=pl.BlockSpec((1, 1, 1), lambda i: (i, 0, 0)),
    out_shape=jax.ShapeDtypeStruct((_NB_TC, 1, 1), jnp.float32),
)


def kernel(prob, target, weight):
    w_pad = jnp.zeros((_WPAD,), jnp.float32).at[:_C].set(weight)
    sc_part = _nll_sc_partials(prob, target, w_pad)
    tgt_3d = target.reshape(_N // _BR, 1, _BR)
    tc_part = _nll_tc_partials(prob, tgt_3d, weight.reshape(1, _C))
    return -(jnp.sum(sc_part) + jnp.sum(tc_part))
